# baseline (speedup 1.0000x reference)
import functools

import jax
import jax.numpy as jnp
from jax import lax
from jax.experimental import pallas as pl
from jax.experimental.pallas import tpu as pltpu
from jax.experimental.pallas import tpu_sc as plsc

NC, NS, L = 2, 16, 16
NW = NC * NS
ROWS = 20000
RPW = 640
BLKS = RPW // L
GCH = 128
NG = RPW // GCH
H = W = 1024
HW = H * W

_mesh = plsc.VectorSubcoreMesh(core_axis_name="c", subcore_axis_name="s")


@functools.partial(
    pl.kernel,
    mesh=_mesh,
    out_type=jax.ShapeDtypeStruct((ROWS * 8,), jnp.float32),
    scratch_types=[
        pltpu.VMEM((RPW * 7,), jnp.float32),
        pltpu.VMEM((RPW * 8,), jnp.float32),
        pltpu.VMEM((RPW,), jnp.int32),
        pltpu.VMEM((RPW,), jnp.float32),
        pltpu.SemaphoreType.DMA,
    ],
    compiler_params=pltpu.CompilerParams(needs_layout_passes=False),
)
def _bbox_depth(bflat_hbm, dmt_hbm, out_hbm, bbuf, obuf, ibuf, dbuf, sem):
    wid = lax.axis_index("s") * NC + lax.axis_index("c")
    base = jnp.minimum(wid * RPW, ROWS - RPW)
    pltpu.sync_copy(bflat_hbm.at[pl.ds(base * 7, RPW * 7)], bbuf)
    lanes = lax.iota(jnp.int32, L)
    for r in range(BLKS):
        rows7 = (lanes + (r * L)) * 7
        rows8 = (lanes + (r * L)) * 8
        f = [plsc.load_gather(bbuf, [rows7 + c]) for c in range(7)]
        bid = jnp.clip(f[0].astype(jnp.int32), 0, 15)
        x1 = (f[3] * W).astype(jnp.int32)
        y1 = (f[4] * H).astype(jnp.int32)
        x2 = (f[5] * W).astype(jnp.int32)
        y2 = (f[6] * H).astype(jnp.int32)
        cx = jnp.clip(lax.shift_right_arithmetic(x1 + x2, 1), 0, W - 1)
        cy = jnp.clip(lax.shift_right_arithmetic(y1 + y2, 1), 0, H - 1)
        ibuf[pl.ds(r * L, L)] = (
            bid * HW
            + lax.shift_right_arithmetic(cy, 3) * 8192
            + lax.shift_right_arithmetic(cx, 7) * 1024
            + lax.bitwise_and(cy, 7) * 128
            + lax.bitwise_and(cx, 127)
        )
        for c in range(7):
            plsc.store_scatter(obuf, [rows8 + c], f[c])
    copies = [
        pltpu.async_copy(
            dmt_hbm.at[ibuf.at[pl.ds(g * GCH, GCH)]],
            dbuf.at[pl.ds(g * GCH, GCH)],
            sem,
        )
        for g in range(NG)
    ]
    for cp in copies:
        cp.wait()
    for r in range(BLKS):
        rows8 = (lanes + (r * L)) * 8
        plsc.store_scatter(obuf, [rows8 + 7], dbuf[pl.ds(r * L, L)])
    pltpu.sync_copy(obuf, out_hbm.at[pl.ds(base * 8, RPW * 8)])


def kernel(bboxes, depth_map):
    bflat = bboxes.reshape(ROWS * 7)
    dmt = (
        depth_map.reshape(16, 128, 8, 8, 128)
        .transpose(0, 1, 3, 2, 4)
        .reshape(16 * HW)
    )
    return _bbox_depth(bflat, dmt).reshape(ROWS, 8)
